# baseline (speedup 1.0000x reference)
import jax
import jax.numpy as jnp
from jax.experimental import pallas as pl
from jax.experimental.pallas import tpu as pltpu


def _gcn(adj, h, w):
    return jax.nn.relu(jnp.einsum('bnm,bmd->bnd', adj, h @ w))


def _level1_kernel(adj_ref, x_ref, w1_ref, w2_ref, p1_ref, p2_ref,
                   z1max_ref, x2_ref, adj2_ref):
    adj = adj_ref[0]
    xb = x_ref[0]
    f32 = jnp.float32
    h1 = jnp.dot(xb, w1_ref[...], preferred_element_type=f32)
    z1a = jnp.maximum(jnp.dot(adj, h1, preferred_element_type=f32), 0.0)
    hp = jnp.dot(xb, p1_ref[...], preferred_element_type=f32)
    sh = jnp.maximum(jnp.dot(adj, hp, preferred_element_type=f32), 0.0)
    u = jnp.concatenate(
        [jnp.dot(z1a, w2_ref[...], preferred_element_type=f32),
         jnp.dot(sh, p2_ref[...], preferred_element_type=f32)], axis=1)
    y = jnp.dot(adj, u, preferred_element_type=f32)
    z1 = jnp.maximum(y[:, :64], 0.0)
    logits = y[:, 64:]
    m = jnp.max(logits, axis=1, keepdims=True)
    e = jnp.exp(logits - m)
    s1 = e / jnp.sum(e, axis=1, keepdims=True)
    t = jnp.dot(adj, s1, preferred_element_type=f32)
    dn = (((0,), (0,)), ((), ()))
    x2 = jax.lax.dot_general(s1, z1, dn, preferred_element_type=f32)
    adj2 = jax.lax.dot_general(s1, t, dn, preferred_element_type=f32)
    z1max_ref[0] = jnp.max(z1, axis=0, keepdims=True)
    x2_ref[0] = x2
    adj2_ref[0] = adj2


def _level1(adj, x, W1, W2, P1, P2, interpret=False):
    B, N, _ = adj.shape
    f_in = x.shape[2]
    n_hid = P2.shape[1]
    out_shapes = (
        jax.ShapeDtypeStruct((B, 1, 64), jnp.float32),
        jax.ShapeDtypeStruct((B, n_hid, 64), jnp.float32),
        jax.ShapeDtypeStruct((B, n_hid, n_hid), jnp.float32),
    )
    return pl.pallas_call(
        _level1_kernel,
        grid=(B,),
        in_specs=[
            pl.BlockSpec((1, N, N), lambda b: (b, 0, 0)),
            pl.BlockSpec((1, N, f_in), lambda b: (b, 0, 0)),
            pl.BlockSpec(W1.shape, lambda b: (0, 0)),
            pl.BlockSpec(W2.shape, lambda b: (0, 0)),
            pl.BlockSpec(P1.shape, lambda b: (0, 0)),
            pl.BlockSpec(P2.shape, lambda b: (0, 0)),
        ],
        out_specs=(
            pl.BlockSpec((1, 1, 64), lambda b: (b, 0, 0)),
            pl.BlockSpec((1, n_hid, 64), lambda b: (b, 0, 0)),
            pl.BlockSpec((1, n_hid, n_hid), lambda b: (b, 0, 0)),
        ),
        out_shape=out_shapes,
        compiler_params=pltpu.CompilerParams(
            dimension_semantics=("parallel",),
        ),
        interpret=interpret,
    )(adj, x, W1, W2, P1, P2)


def kernel(x, adj, W1, W2, P1, P2, W3, W4, P3, P4, W5, W6):
    z1max, x2, adj2 = _level1(adj, x, W1, W2, P1, P2)
    z2 = _gcn(adj2, x2, W3)
    z2 = _gcn(adj2, z2, W4)
    sh2 = _gcn(adj2, x2, P3)
    s2 = jax.nn.softmax(jnp.einsum('bnm,bmd->bnd', adj2, sh2 @ P4), axis=-1)
    x3 = jnp.einsum('bnc,bnd->bcd', s2, z2)
    adj3 = jnp.einsum('bnc,bnm,bmk->bck', s2, adj2, s2)
    z3 = _gcn(adj3, x3, W5)
    z3 = _gcn(adj3, z3, W6)
    emb = jnp.concatenate(
        [z1max[:, 0, :], z2.max(axis=1), z3.max(axis=1)], axis=-1)
    g = emb.reshape(emb.shape[0], 1, emb.shape[1])
    return jax.nn.relu(g)


# speedup vs baseline: 1.2344x; 1.2344x over previous
import jax
import jax.numpy as jnp
from jax.experimental import pallas as pl
from jax.experimental.pallas import tpu as pltpu


def _gcn(adj, h, w):
    return jax.nn.relu(jnp.einsum('bnm,bmd->bnd', adj, h @ w))


def _level1_kernel(adj_ref, x_ref, w1_ref, w2_ref, p1_ref, p2_ref,
                   z1max_ref, x2_ref, adj2_ref):
    adj = adj_ref[0]
    xb = x_ref[0]
    f32 = jnp.float32
    c1 = jnp.concatenate([w1_ref[...], p1_ref[...]], axis=1)
    h = jnp.dot(xb, c1, preferred_element_type=f32)
    g1 = jnp.maximum(jnp.dot(adj, h, preferred_element_type=f32), 0.0)
    g2 = jnp.dot(adj, g1, preferred_element_type=f32)
    z1 = jnp.maximum(
        jnp.dot(g2[:, :64], w2_ref[...], preferred_element_type=f32), 0.0)
    logits = jnp.dot(g2[:, 64:], p2_ref[...], preferred_element_type=f32)
    m = jnp.max(logits, axis=1, keepdims=True)
    e = jnp.exp(logits - m)
    s1 = e / jnp.sum(e, axis=1, keepdims=True)
    t = jnp.dot(adj, s1, preferred_element_type=f32)
    dn = (((0,), (0,)), ((), ()))
    x2 = jax.lax.dot_general(s1, z1, dn, preferred_element_type=f32)
    adj2 = jax.lax.dot_general(s1, t, dn, preferred_element_type=f32)
    z1max_ref[0] = jnp.max(z1, axis=0, keepdims=True)
    x2_ref[0] = x2
    adj2_ref[0] = adj2


def _level1(adj, x, W1, W2, P1, P2, interpret=False):
    B, N, _ = adj.shape
    f_in = x.shape[2]
    n_hid = P2.shape[1]
    out_shapes = (
        jax.ShapeDtypeStruct((B, 1, 64), jnp.float32),
        jax.ShapeDtypeStruct((B, n_hid, 64), jnp.float32),
        jax.ShapeDtypeStruct((B, n_hid, n_hid), jnp.float32),
    )
    return pl.pallas_call(
        _level1_kernel,
        grid=(B,),
        in_specs=[
            pl.BlockSpec((1, N, N), lambda b: (b, 0, 0)),
            pl.BlockSpec((1, N, f_in), lambda b: (b, 0, 0)),
            pl.BlockSpec(W1.shape, lambda b: (0, 0)),
            pl.BlockSpec(W2.shape, lambda b: (0, 0)),
            pl.BlockSpec(P1.shape, lambda b: (0, 0)),
            pl.BlockSpec(P2.shape, lambda b: (0, 0)),
        ],
        out_specs=(
            pl.BlockSpec((1, 1, 64), lambda b: (b, 0, 0)),
            pl.BlockSpec((1, n_hid, 64), lambda b: (b, 0, 0)),
            pl.BlockSpec((1, n_hid, n_hid), lambda b: (b, 0, 0)),
        ),
        out_shape=out_shapes,
        compiler_params=pltpu.CompilerParams(
            dimension_semantics=("parallel",),
        ),
        interpret=interpret,
    )(adj, x, W1, W2, P1, P2)


def kernel(x, adj, W1, W2, P1, P2, W3, W4, P3, P4, W5, W6):
    z1max, x2, adj2 = _level1(adj, x, W1, W2, P1, P2)
    z2 = _gcn(adj2, x2, W3)
    z2 = _gcn(adj2, z2, W4)
    sh2 = _gcn(adj2, x2, P3)
    s2 = jax.nn.softmax(jnp.einsum('bnm,bmd->bnd', adj2, sh2 @ P4), axis=-1)
    x3 = jnp.einsum('bnc,bnd->bcd', s2, z2)
    adj3 = jnp.einsum('bnc,bnm,bmk->bck', s2, adj2, s2)
    z3 = _gcn(adj3, x3, W5)
    z3 = _gcn(adj3, z3, W6)
    emb = jnp.concatenate(
        [z1max[:, 0, :], z2.max(axis=1), z3.max(axis=1)], axis=-1)
    g = emb.reshape(emb.shape[0], 1, emb.shape[1])
    return jax.nn.relu(g)
